# baseline (device time: 41729 ns/iter reference)
import numpy as np
import jax
import jax.numpy as jnp
from jax import lax
from jax.experimental import pallas as pl
from jax.experimental.pallas import tpu as pltpu

N_DEV = 16
B, SQ, DM = 2, 256, 768
HQ_SHARD = 4
DH = 64
HD_SHARD = HQ_SHARD * DH
CH = SQ // N_DEV


def _tables():
    inv = 1.0 / (10000.0 ** (np.arange(0, DH, 2) / DH))
    pos = np.arange(SQ)[:, None] * inv[None, :]
    cos = np.repeat(np.cos(pos), 2, axis=-1)
    sin = np.repeat(np.sin(pos), 2, axis=-1)
    cos_t = np.tile(cos, (1, HQ_SHARD)).astype(np.float32)
    sin_t = np.tile(sin, (1, HQ_SHARD)).astype(np.float32)
    P = np.zeros((HD_SHARD, HD_SHARD), np.float32)
    for c in range(0, HD_SHARD, 2):
        P[c + 1, c] = -1.0
        P[c, c + 1] = 1.0
    return cos_t, sin_t, P


def kernel(x, Wq, Wk, Wv, Wo):
    cos_t, sin_t, P = _tables()
    f32 = jnp.float32
    bf16 = jnp.bfloat16

    def body(x_ref, wq_ref, wk_ref, wv_ref, wo_ref, cos_ref, sin_ref, p_ref,
             out_ref, partial_ref, reduced_ref, ctx_ref, acc_ref,
             rs_buf, ag_buf, rs_send, rs_recv, ag_send, ag_recv):
        my_d = lax.axis_index("i")
        cos = cos_ref[:, :]
        sin = sin_ref[:, :]
        pmat = p_ref[:, :].astype(bf16)
        wo16 = wo_ref[:, :].astype(bf16)
        wq16 = wq_ref[:, :].astype(bf16)
        wk16 = wk_ref[:, :].astype(bf16)
        wv16 = wv_ref[:, :].astype(bf16)

        def attention(b):
            xb = x_ref[b].astype(bf16)
            q = jnp.dot(xb, wq16, preferred_element_type=f32)
            k = jnp.dot(xb, wk16, preferred_element_type=f32)
            v = jnp.dot(xb, wv16, preferred_element_type=f32)
            q = q * cos + jnp.dot(q.astype(bf16), pmat,
                                  preferred_element_type=f32) * sin
            k = k * cos + jnp.dot(k.astype(bf16), pmat,
                                  preferred_element_type=f32) * sin
            q16, k16, v16 = q.astype(bf16), k.astype(bf16), v.astype(bf16)
            for h in range(HQ_SHARD):
                sl = slice(h * DH, (h + 1) * DH)
                s = lax.dot_general(q16[:, sl], k16[:, sl],
                                    (((1,), (1,)), ((), ())),
                                    preferred_element_type=f32) * 0.125
                m = jnp.max(s, axis=-1, keepdims=True)
                w = jnp.exp(s - m)
                w = w / jnp.sum(w, axis=-1, keepdims=True)
                ctx_ref[b, :, sl] = jnp.dot(w.astype(bf16), v16[:, sl],
                                            preferred_element_type=f32
                                            ).astype(bf16)

        rs_rdmas = {}
        ag_rdmas = {}

        def rs_send_batch(b):
            for k in range(1, N_DEV):
                t = lax.rem(my_d + k, N_DEV)
                pc = jnp.dot(ctx_ref[b, pl.ds(t * CH, CH), :], wo16,
                             preferred_element_type=f32)
                partial_ref[b, pl.ds(t * CH, CH), :] = pc.astype(bf16)
                rdma = pltpu.make_async_remote_copy(
                    src_ref=partial_ref.at[b, pl.ds(t * CH, CH), :],
                    dst_ref=rs_buf.at[k, b],
                    send_sem=rs_send.at[k, b],
                    recv_sem=rs_recv.at[k, b],
                    device_id=(t,),
                    device_id_type=pl.DeviceIdType.MESH,
                )
                rdma.start()
                rs_rdmas[(k, b)] = rdma
            acc_ref[b] = jnp.dot(ctx_ref[b, pl.ds(my_d * CH, CH), :], wo16,
                                 preferred_element_type=f32)

        def reduce_and_ag_send(b):
            for k in range(1, N_DEV):
                rs_rdmas[(k, b)].wait_recv()
                acc_ref[b] = acc_ref[b] + rs_buf[k, b].astype(f32)
            accb = acc_ref[b]
            reduced_ref[b] = accb.astype(bf16)
            out_ref[b, pl.ds(my_d * CH, CH), :] = accb
            for k in range(1, N_DEV):
                t = lax.rem(my_d + k, N_DEV)
                rdma = pltpu.make_async_remote_copy(
                    src_ref=reduced_ref.at[b],
                    dst_ref=ag_buf.at[k, b],
                    send_sem=ag_send.at[k, b],
                    recv_sem=ag_recv.at[k, b],
                    device_id=(t,),
                    device_id_type=pl.DeviceIdType.MESH,
                )
                rdma.start()
                ag_rdmas[(k, b)] = rdma

        def ag_recv_batch(b):
            for k in range(1, N_DEV):
                ag_rdmas[(k, b)].wait_recv()
                c = lax.rem(my_d - k + N_DEV, N_DEV)
                out_ref[b, pl.ds(c * CH, CH), :] = ag_buf[k, b].astype(f32)

        attention(0)
        rs_send_batch(0)
        attention(1)
        rs_send_batch(1)
        reduce_and_ag_send(0)
        reduce_and_ag_send(1)
        ag_recv_batch(0)
        ag_recv_batch(1)

        for r in rs_rdmas.values():
            r.wait_send()
        for r in ag_rdmas.values():
            r.wait_send()

    return pl.pallas_call(
        body,
        out_shape=jax.ShapeDtypeStruct((B, SQ, DM), f32),
        in_specs=[pl.BlockSpec(memory_space=pltpu.VMEM)] * 8,
        out_specs=pl.BlockSpec(memory_space=pltpu.VMEM),
        scratch_shapes=[
            pltpu.VMEM((B, SQ, DM), bf16),
            pltpu.VMEM((B, CH, DM), bf16),
            pltpu.VMEM((B, SQ, HD_SHARD), bf16),
            pltpu.VMEM((B, CH, DM), f32),
            pltpu.VMEM((N_DEV, B, CH, DM), bf16),
            pltpu.VMEM((N_DEV, B, CH, DM), bf16),
            pltpu.SemaphoreType.DMA((N_DEV, B)),
            pltpu.SemaphoreType.DMA((N_DEV, B)),
            pltpu.SemaphoreType.DMA((N_DEV, B)),
            pltpu.SemaphoreType.DMA((N_DEV, B)),
        ],
    )(x, Wq, Wk, Wv, Wo, jnp.asarray(cos_t), jnp.asarray(sin_t),
      jnp.asarray(P))


# device time: 40419 ns/iter; 1.0324x vs baseline; 1.0324x over previous
import numpy as np
import jax
import jax.numpy as jnp
from jax import lax
from jax.experimental import pallas as pl
from jax.experimental.pallas import tpu as pltpu

N_DEV = 16
B, SQ, DM = 2, 256, 768
HQ_SHARD = 4
DH = 64
HD_SHARD = HQ_SHARD * DH
CH = SQ // N_DEV


def _tables():
    inv = 1.0 / (10000.0 ** (np.arange(0, DH, 2) / DH))
    pos = np.arange(SQ)[:, None] * inv[None, :]
    cos = np.repeat(np.cos(pos), 2, axis=-1)
    sin = np.repeat(np.sin(pos), 2, axis=-1)
    cos_t = np.tile(cos, (1, HQ_SHARD)).astype(np.float32)
    sin_t = np.tile(sin, (1, HQ_SHARD)).astype(np.float32)
    P = np.zeros((HD_SHARD, HD_SHARD), np.float32)
    for c in range(0, HD_SHARD, 2):
        P[c + 1, c] = -1.0
        P[c, c + 1] = 1.0
    return cos_t, sin_t, P


def kernel(x, Wq, Wk, Wv, Wo):
    cos_t, sin_t, P = _tables()
    f32 = jnp.float32
    bf16 = jnp.bfloat16

    def body(x_ref, wq_ref, wk_ref, wv_ref, wo_ref, cos_ref, sin_ref, p_ref,
             out_ref, partial_ref, pf32_ref, reduced_ref, ctx_ref, acc_ref,
             rs_buf, ag_buf, rs_send, rs_recv, ag_send, ag_recv):
        my_d = lax.axis_index("i")
        cos = cos_ref[:, :]
        sin = sin_ref[:, :]
        pmat = p_ref[:, :].astype(bf16)
        wo16 = wo_ref[:, :].astype(bf16)

        wq16 = wq_ref[:, :].astype(bf16)
        wk16 = wk_ref[:, :].astype(bf16)
        wv16 = wv_ref[:, :].astype(bf16)
        for b in range(B):
            xb = x_ref[b].astype(bf16)
            q = jnp.dot(xb, wq16, preferred_element_type=f32)
            k = jnp.dot(xb, wk16, preferred_element_type=f32)
            v = jnp.dot(xb, wv16, preferred_element_type=f32)
            q = q * cos + jnp.dot(q.astype(bf16), pmat,
                                  preferred_element_type=f32) * sin
            k = k * cos + jnp.dot(k.astype(bf16), pmat,
                                  preferred_element_type=f32) * sin
            q16 = (q * 0.125).astype(bf16)
            k16, v16 = k.astype(bf16), v.astype(bf16)
            for h in range(HQ_SHARD):
                sl = slice(h * DH, (h + 1) * DH)
                s = lax.dot_general(q16[:, sl], k16[:, sl],
                                    (((1,), (1,)), ((), ())),
                                    preferred_element_type=f32)
                w = jnp.exp(s)
                denom = jnp.sum(w, axis=-1, keepdims=True)
                ctx = jnp.dot(w.astype(bf16), v16[:, sl],
                              preferred_element_type=f32)
                ctx_ref[b, :, sl] = (ctx / denom).astype(bf16)

        for b in range(B):
            pb = jnp.dot(ctx_ref[b], wo16, preferred_element_type=f32)
            pf32_ref[b] = pb
            partial_ref[b] = pb.astype(bf16)

        rs_rdmas = []
        for k in range(1, N_DEV):
            t = lax.rem(my_d + k, N_DEV)
            rdma = pltpu.make_async_remote_copy(
                src_ref=partial_ref.at[:, pl.ds(t * CH, CH), :],
                dst_ref=rs_buf.at[k],
                send_sem=rs_send.at[k],
                recv_sem=rs_recv.at[k],
                device_id=(t,),
                device_id_type=pl.DeviceIdType.MESH,
            )
            rdma.start()
            rs_rdmas.append(rdma)

        for b in range(B):
            acc_ref[b] = pf32_ref[b, pl.ds(my_d * CH, CH), :]
        for k in range(1, N_DEV):
            rs_rdmas[k - 1].wait_recv()
            acc_ref[:, :, :] = acc_ref[:, :, :] + rs_buf[k].astype(f32)

        acc = acc_ref[:, :, :]
        reduced_ref[:, :, :] = acc.astype(bf16)
        out_ref[:, pl.ds(my_d * CH, CH), :] = acc

        ag_rdmas = []
        for k in range(1, N_DEV):
            t = lax.rem(my_d + k, N_DEV)
            rdma = pltpu.make_async_remote_copy(
                src_ref=reduced_ref,
                dst_ref=ag_buf.at[k],
                send_sem=ag_send.at[k],
                recv_sem=ag_recv.at[k],
                device_id=(t,),
                device_id_type=pl.DeviceIdType.MESH,
            )
            rdma.start()
            ag_rdmas.append(rdma)

        for k in range(1, N_DEV):
            ag_rdmas[k - 1].wait_recv()
            c = lax.rem(my_d - k + N_DEV, N_DEV)
            out_ref[:, pl.ds(c * CH, CH), :] = ag_buf[k].astype(f32)

        for r in rs_rdmas:
            r.wait_send()
        for r in ag_rdmas:
            r.wait_send()

    return pl.pallas_call(
        body,
        out_shape=jax.ShapeDtypeStruct((B, SQ, DM), f32),
        in_specs=[pl.BlockSpec(memory_space=pltpu.VMEM)] * 8,
        out_specs=pl.BlockSpec(memory_space=pltpu.VMEM),
        scratch_shapes=[
            pltpu.VMEM((B, SQ, DM), bf16),
            pltpu.VMEM((B, SQ, DM), f32),
            pltpu.VMEM((B, CH, DM), bf16),
            pltpu.VMEM((B, SQ, HD_SHARD), bf16),
            pltpu.VMEM((B, CH, DM), f32),
            pltpu.VMEM((N_DEV, B, CH, DM), bf16),
            pltpu.VMEM((N_DEV, B, CH, DM), bf16),
            pltpu.SemaphoreType.DMA((N_DEV,)),
            pltpu.SemaphoreType.DMA((N_DEV,)),
            pltpu.SemaphoreType.DMA((N_DEV,)),
            pltpu.SemaphoreType.DMA((N_DEV,)),
        ],
    )(x, Wq, Wk, Wv, Wo, jnp.asarray(cos_t), jnp.asarray(sin_t),
      jnp.asarray(P))
